# baseline (device time: 10441 ns/iter reference)
import jax
import jax.numpy as jnp
from jax import lax
from jax.experimental import pallas as pl
from jax.experimental.pallas import tpu as pltpu

N_DEV = 16
N_GLOBAL_ROWS = 16384


def kernel(x):
    m_per, n = x.shape

    def body(
        x_hbm_ref, out_ref, x_vmem, comm_ref, copy_sem,
        ready_sems, send_sems, recv_sems,
    ):
        my_pos = lax.axis_index("i")

        for e in range(1, N_DEV):
            peer = lax.rem(my_pos + e, N_DEV)
            pl.semaphore_signal(
                ready_sems.at[N_DEV - e], inc=1,
                device_id=(peer,), device_id_type=pl.DeviceIdType.MESH,
            )

        barrier_sem = pltpu.get_barrier_semaphore()
        pl.semaphore_signal(barrier_sem, inc=1)
        pl.semaphore_wait(barrier_sem, 1)

        cp = pltpu.make_async_copy(x_hbm_ref, x_vmem, copy_sem)
        cp.start()
        cp.wait()
        comm_ref[pl.ds(0, 1), :] = jnp.sum(x_vmem[:, :], axis=0, keepdims=True)

        rdmas = []
        for d in range(1, N_DEV):
            peer = lax.rem(my_pos + d, N_DEV)
            rdma = pltpu.make_async_remote_copy(
                src_ref=comm_ref.at[pl.ds(0, 1)],
                dst_ref=comm_ref.at[pl.ds(d, 1)],
                send_sem=send_sems.at[d - 1],
                recv_sem=recv_sems.at[d - 1],
                device_id=(peer,),
                device_id_type=pl.DeviceIdType.MESH,
            )
            pl.semaphore_wait(ready_sems.at[d], 1)
            rdma.start()
            rdmas.append(rdma)

        for rdma in rdmas:
            rdma.wait()

        out_ref[:, :] = jnp.sum(comm_ref[:, :], axis=0, keepdims=True) * (
            1.0 / N_GLOBAL_ROWS
        )

    return pl.pallas_call(
        body,
        out_shape=jax.ShapeDtypeStruct((1, n), jnp.float32),
        in_specs=[pl.BlockSpec(memory_space=pl.ANY)],
        out_specs=pl.BlockSpec(memory_space=pltpu.VMEM),
        scratch_shapes=[
            pltpu.VMEM((m_per, n), jnp.float32),
            pltpu.VMEM((N_DEV, n), jnp.float32),
            pltpu.SemaphoreType.DMA(()),
            pltpu.SemaphoreType.REGULAR((N_DEV,)),
            pltpu.SemaphoreType.DMA((N_DEV - 1,)),
            pltpu.SemaphoreType.DMA((N_DEV - 1,)),
        ],
        compiler_params=pltpu.CompilerParams(collective_id=0),
    )(x)
